# baseline (speedup 1.0000x reference)
import functools

import jax
import jax.numpy as jnp
from jax import lax
from jax.experimental import pallas as pl
from jax.experimental.pallas import tpu as pltpu
from jax.experimental.pallas import tpu_sc as plsc

NEMB = 16
NVEC = 32
NSTEPS = 10
NNODES = 10000
NEDGES = 160000
NGRAPHS = 100
NSTEMS = 2000
NBLK = 106
NST = 21
NBOND = 20
OUT_PER_STEM = 105

NC, NS = 2, 16
NW = NC * NS
EPT = 5120
EPAD = NW * EPT
TBINS = NBOND * NNODES
TPAD = TBINS + 192
ZCH = TPAD // NS
PTAIL_BASE = (NS - 1) * ZCH
PTAIL = TBINS - PTAIL_BASE
NCK = 4
ECH = EPT // NCK
NSCAL = NEMB * 2048
SPT = NSCAL // NW

_lrelu = lambda t: jnp.where(t >= 0, t, 0.01 * t)
_f32 = jnp.float32


def _dot(a, b):
    return jnp.dot(a, b, preferred_element_type=_f32)


def _edge_body(p_hbm, gix_hbm, cix_hbm, tout_hbm,
               gix0, gix1, gix2, gix3, cix0, cix1, cix2, cix3,
               val0, val1, val2, val3, zbuf_v, tsh, psh, sem):
    gix_v = [gix0, gix1, gix2, gix3]
    cix_v = [cix0, cix1, cix2, cix3]
    vals_v = [val0, val1, val2, val3]
    cid = lax.axis_index("c")
    sid = lax.axis_index("s")
    wid = cid * NS + sid
    ebase = pl.multiple_of(wid * EPT, EPT)
    zbase = pl.multiple_of(sid * ZCH, ZCH)

    @pl.when(sid < NS - 1)
    def _stage_full():
        pltpu.sync_copy(p_hbm.at[pl.ds(zbase, ZCH)], zbuf_v)
        pltpu.sync_copy(zbuf_v, psh.at[pl.ds(zbase, ZCH)])

    @pl.when(sid == NS - 1)
    def _stage_tail():
        pltpu.sync_copy(p_hbm.at[pl.ds(PTAIL_BASE, PTAIL)],
                        zbuf_v.at[pl.ds(0, PTAIL)])
        pltpu.sync_copy(zbuf_v.at[pl.ds(0, PTAIL)],
                        psh.at[pl.ds(PTAIL_BASE, PTAIL)])

    def _z(i, carry):
        zbuf_v[pl.ds(pl.multiple_of(i * 16, 16), 16)] = jnp.zeros((16,), _f32)
        return carry
    lax.fori_loop(0, ZCH // 16, _z, 0, unroll=8)
    pltpu.sync_copy(zbuf_v, tsh.at[pl.ds(zbase, ZCH)])
    for j in range(NCK):
        pltpu.sync_copy(gix_hbm.at[pl.ds(ebase + j * ECH, ECH)], gix_v[j])
        pltpu.sync_copy(cix_hbm.at[pl.ds(ebase + j * ECH, ECH)], cix_v[j])
    plsc.subcore_barrier()
    gathers = [
        pltpu.make_async_copy(psh.at[gix_v[j]], vals_v[j], sem)
        for j in range(NCK)
    ]
    gathers[0].start()
    for j in range(NCK):
        gathers[j].wait()
        if j + 1 < NCK:
            gathers[j + 1].start()
        pltpu.sync_copy(vals_v[j], tsh.at[cix_v[j]], add=True)
    plsc.subcore_barrier()
    pltpu.sync_copy(tsh.at[pl.ds(zbase, ZCH)], zbuf_v)
    pltpu.sync_copy(zbuf_v, tout_hbm.at[pl.ds(cid * TPAD + zbase, ZCH)])


_edge_pass = pl.kernel(
    _edge_body,
    out_type=jax.ShapeDtypeStruct((NC * TPAD,), _f32),
    mesh=plsc.VectorSubcoreMesh(core_axis_name="c", subcore_axis_name="s",
                                num_cores=NC, num_subcores=NS),
    scratch_types=(
        [pltpu.VMEM((ECH,), jnp.int32)] * (2 * NCK)
        + [pltpu.VMEM((ECH,), _f32)] * NCK
    ) + [
        pltpu.VMEM((ZCH,), _f32),
        pltpu.VMEM_SHARED((TPAD,), _f32),
        pltpu.VMEM_SHARED((TBINS,), _f32),
        pltpu.SemaphoreType.DMA,
    ],
)


def _stem_body(tab_hbm, idx_hbm, out_hbm, idx_v, vals_v, sem):
    cid = lax.axis_index("c")
    sid = lax.axis_index("s")
    wid = cid * NS + sid
    base = pl.multiple_of(wid * SPT, SPT)
    pltpu.sync_copy(idx_hbm.at[pl.ds(base, SPT)], idx_v)
    pltpu.async_copy(tab_hbm.at[idx_v], vals_v, sem).wait()
    pltpu.sync_copy(vals_v, out_hbm.at[pl.ds(base, SPT)])


_stem_gather = pl.kernel(
    _stem_body,
    out_type=jax.ShapeDtypeStruct((NSCAL,), _f32),
    mesh=plsc.VectorSubcoreMesh(core_axis_name="c", subcore_axis_name="s",
                                num_cores=NC, num_subcores=NS),
    scratch_types=[
        pltpu.VMEM((SPT,), jnp.int32),
        pltpu.VMEM((SPT,), _f32),
        pltpu.SemaphoreType.DMA,
    ],
)


def _init_body(xT_ref, batchT_ref, t0_ref, t1_ref, vecT_ref, bembT_blk_ref,
               w1aT_ref, w1bT_ref, b1_ref, w2T_ref, b2_ref, bemb_ref,
               h_ref, p_ref, invdeg_ref):
    oh_x = (xT_ref[...] == lax.broadcasted_iota(jnp.int32, (NBLK, NNODES), 0)
            ).astype(_f32)
    xw = _dot(w1aT_ref[...], bembT_blk_ref[...])
    oh_b = (batchT_ref[...] ==
            lax.broadcasted_iota(jnp.int32, (NGRAPHS, NNODES), 0)).astype(_f32)
    vb = _dot(w1bT_ref[...], vecT_ref[...])
    pre = _dot(xw, oh_x) + _dot(vb, oh_b) + b1_ref[...]
    h0 = _dot(w2T_ref[...], _lrelu(pre)) + b2_ref[...]
    h_ref[...] = h0
    p_ref[...] = _dot(bemb_ref[...], h0)
    deg = jnp.sum(t0_ref[...] + t1_ref[...], axis=0, keepdims=True)
    invdeg_ref[...] = 1.0 / jnp.maximum(deg, 1.0)


def _tc_init(xT, batchT, t0, t1, vecT, bembT_blk, w1aT, w1bT, b1, w2T, b2,
             bemb):
    return pl.pallas_call(
        _init_body,
        out_shape=(
            jax.ShapeDtypeStruct((NEMB, NNODES), _f32),
            jax.ShapeDtypeStruct((NBOND, NNODES), _f32),
            jax.ShapeDtypeStruct((1, NNODES), _f32),
        ),
    )(xT, batchT, t0, t1, vecT, bembT_blk, w1aT, w1bT, b1, w2T, b2, bemb)


def _step_body(t0_ref, t1_ref, invdeg_ref, h_ref, bembT_ref, rootT_ref,
               cb_ref, wihT_ref, bih_ref, whhT_ref, bhh_ref, bemb_ref,
               hout_ref, pout_ref):
    h = h_ref[...]
    t = t0_ref[...] + t1_ref[...]
    aggr = _dot(bembT_ref[...], t) * invdeg_ref[...]
    m = _lrelu(aggr + _dot(rootT_ref[...], h) + cb_ref[...])
    gi = _dot(wihT_ref[...], m) + bih_ref[...]
    gh = _dot(whhT_ref[...], h) + bhh_ref[...]
    r = jax.nn.sigmoid(gi[0:NEMB] + gh[0:NEMB])
    z = jax.nn.sigmoid(gi[NEMB:2 * NEMB] + gh[NEMB:2 * NEMB])
    n = jnp.tanh(gi[2 * NEMB:] + r * gh[2 * NEMB:])
    hn = (1.0 - z) * n + z * h
    hout_ref[...] = hn
    pout_ref[...] = _dot(bemb_ref[...], hn)


def _tc_step(t0, t1, invdeg, h, bembT, rootT, cb, wihT, bih, whhT, bhh, bemb):
    return pl.pallas_call(
        _step_body,
        out_shape=(
            jax.ShapeDtypeStruct((NEMB, NNODES), _f32),
            jax.ShapeDtypeStruct((NBOND, NNODES), _f32),
        ),
    )(t0, t1, invdeg, h, bembT, rootT, cb, wihT, bih, whhT, bhh, bemb)


def _head_body(rows_ref, stypesT_ref, batchT_ref, h_ref, stembT_ref,
               sw1aT_ref, sw1bT_ref, sb1_ref, sw2T_ref, sb2_ref, sw3_ref,
               sb3_ref, gw1_ref, gb1_ref, gw2_ref, gb2_ref,
               sp_ref, mp_ref):
    oh_st = (stypesT_ref[...] ==
             lax.broadcasted_iota(jnp.int32, (NST, 2048), 0)).astype(_f32)
    st = _dot(stembT_ref[...], oh_st)
    hs = _lrelu(_dot(sw1aT_ref[...], rows_ref[...])
                + _dot(sw1bT_ref[...], st) + sb1_ref[...])
    hs = _lrelu(_dot(sw2T_ref[...], hs) + sb2_ref[...])
    sp_ref[...] = lax.dot_general(
        hs, sw3_ref[...], (((0,), (0,)), ((), ())),
        preferred_element_type=_f32) + sb3_ref[...]
    oh_g = (batchT_ref[...] ==
            lax.broadcasted_iota(jnp.int32, (NGRAPHS, NNODES), 0)).astype(_f32)
    gsum = lax.dot_general(
        oh_g, h_ref[...], (((1,), (1,)), ((), ())),
        preferred_element_type=_f32)
    gcnt = jnp.sum(oh_g, axis=1, keepdims=True)
    gmean = gsum / jnp.maximum(gcnt, 1.0)
    gm = _lrelu(_dot(gmean, gw1_ref[...]) + gb1_ref[...])
    mp_ref[...] = _dot(gm, gw2_ref[...]) + gb2_ref[...]


def _tc_head(rows, stypesT, batchT, h, stembT, sw1aT, sw1bT, sb1, sw2T, sb2,
             sw3, sb3, gw1, gb1, gw2, gb2):
    return pl.pallas_call(
        _head_body,
        out_shape=(
            jax.ShapeDtypeStruct((2048, OUT_PER_STEM), _f32),
            jax.ShapeDtypeStruct((NGRAPHS, 1), _f32),
        ),
    )(rows, stypesT, batchT, h, stembT, sw1aT, sw1bT, sb1, sw2T, sb2,
      sw3, sb3, gw1, gb1, gw2, gb2)


def kernel(x, edge_index, edge_attr_idx, stemtypes, batch, stems_batch, stems,
           slices_x, vec_data, blockemb, stememb, bondemb, conv_root,
           conv_bias, b2e_w1, b2e_b1, b2e_w2, b2e_b2, gru_wih, gru_bih,
           gru_whh, gru_bhh, s_w1, s_b1, s_w2, s_b2, s_w3, s_b3, g_w1, g_b1,
           g_w2, g_b2):
    i32 = jnp.int32

    src = edge_index[0].astype(i32)
    dst = edge_index[1].astype(i32)
    i0 = edge_attr_idx[:, 0].astype(i32)
    i1 = edge_attr_idx[:, 1].astype(i32)
    gix = jnp.concatenate(
        [i0 * NNODES + src, jnp.zeros((EPAD - NEDGES,), i32)])
    cix = jnp.concatenate(
        [i1 * NNODES + dst, jnp.full((EPAD - NEDGES,), TBINS, i32)])
    ones_t = jnp.ones((TBINS,), _f32)

    sidx = (jnp.take(slices_x, stems_batch).astype(i32)
            + stems[:, 0].astype(i32))
    sidx_p = jnp.concatenate([sidx, jnp.zeros((2048 - NSTEMS,), i32)])
    sflat = (jnp.arange(NEMB, dtype=i32)[:, None] * NNODES
             + sidx_p[None, :]).reshape(NSCAL)
    stypesT = jnp.concatenate(
        [stemtypes.astype(i32), jnp.zeros((2048 - NSTEMS,), i32)]
    ).reshape(1, 2048)

    xT = x.astype(i32).reshape(1, NNODES)
    batchT = batch.astype(i32).reshape(1, NNODES)

    bembT = jnp.transpose(bondemb)
    w1aT = jnp.transpose(b2e_w1[:NEMB])
    w1bT = jnp.transpose(b2e_w1[NEMB:])
    vecT = jnp.transpose(vec_data)
    bembT_blk = jnp.transpose(blockemb)
    b1 = b2e_b1.reshape(NEMB, 1)
    w2T = jnp.transpose(b2e_w2)
    b2 = b2e_b2.reshape(NEMB, 1)
    rootT = jnp.transpose(conv_root)
    cb = conv_bias.reshape(NEMB, 1)
    wihT = jnp.transpose(gru_wih)
    whhT = jnp.transpose(gru_whh)
    bih = gru_bih.reshape(3 * NEMB, 1)
    bhh = gru_bhh.reshape(3 * NEMB, 1)
    sw1aT = jnp.transpose(s_w1[:NEMB])
    sw1bT = jnp.transpose(s_w1[NEMB:])
    sb1 = s_b1.reshape(NEMB, 1)
    sw2T = jnp.transpose(s_w2)
    sb2 = s_b2.reshape(NEMB, 1)
    sb3 = s_b3.reshape(1, OUT_PER_STEM)
    stembT = jnp.transpose(stememb)
    gb1 = g_b1.reshape(1, NEMB)
    gb2 = g_b2.reshape(1, 1)

    cpart = _edge_pass(ones_t, gix, cix)
    t0c = cpart[:TBINS].reshape(NBOND, NNODES)
    t1c = cpart[TPAD:TPAD + TBINS].reshape(NBOND, NNODES)

    h, p, invdeg = _tc_init(xT, batchT, t0c, t1c, vecT, bembT_blk,
                            w1aT, w1bT, b1, w2T, b2, bondemb)

    for _ in range(NSTEPS):
        tpart = _edge_pass(p.reshape(TBINS), gix, cix)
        t0 = tpart[:TBINS].reshape(NBOND, NNODES)
        t1 = tpart[TPAD:TPAD + TBINS].reshape(NBOND, NNODES)
        h, p = _tc_step(t0, t1, invdeg, h, bembT, rootT, cb,
                        wihT, bih, whhT, bhh, bondemb)

    rows = _stem_gather(h.reshape(NEMB * NNODES), sflat).reshape(NEMB, 2048)
    sp, mp = _tc_head(rows, stypesT, batchT, h, stembT, sw1aT, sw1bT, sb1,
                      sw2T, sb2, s_w3, sb3, g_w1, gb1, g_w2, gb2)
    return (sp[:NSTEMS], mp)


# speedup vs baseline: 1.0499x; 1.0499x over previous
import functools

import jax
import jax.numpy as jnp
from jax import lax
from jax.experimental import pallas as pl
from jax.experimental.pallas import tpu as pltpu
from jax.experimental.pallas import tpu_sc as plsc

NEMB = 16
NVEC = 32
NSTEPS = 10
NNODES = 10000
NEDGES = 160000
NGRAPHS = 100
NSTEMS = 2000
NBLK = 106
NST = 21
NBOND = 20
OUT_PER_STEM = 105

NC, NS = 2, 16
NW = NC * NS
EPT = 5120
EPAD = NW * EPT
TBINS = NBOND * NNODES
TPAD = TBINS + 192
ZCH = TPAD // NS
PTAIL_BASE = (NS - 1) * ZCH
PTAIL = TBINS - PTAIL_BASE
NSCAL = NEMB * 2048
SPT = NSCAL // NW

_lrelu = lambda t: jnp.where(t >= 0, t, 0.01 * t)
_f32 = jnp.float32


def _dot(a, b):
    return jnp.dot(a, b, preferred_element_type=_f32)


def _edge_body(p_hbm, gix_hbm, cix_hbm, tout_hbm,
               gix_v, cix_v, vals_v, zbuf_v, tsh, psh, sem):
    cid = lax.axis_index("c")
    sid = lax.axis_index("s")
    wid = cid * NS + sid
    ebase = pl.multiple_of(wid * EPT, EPT)
    zbase = pl.multiple_of(sid * ZCH, ZCH)

    @pl.when(sid < NS - 1)
    def _stage_full():
        pltpu.sync_copy(p_hbm.at[pl.ds(zbase, ZCH)], zbuf_v)
        pltpu.sync_copy(zbuf_v, psh.at[pl.ds(zbase, ZCH)])

    @pl.when(sid == NS - 1)
    def _stage_tail():
        pltpu.sync_copy(p_hbm.at[pl.ds(PTAIL_BASE, PTAIL)],
                        zbuf_v.at[pl.ds(0, PTAIL)])
        pltpu.sync_copy(zbuf_v.at[pl.ds(0, PTAIL)],
                        psh.at[pl.ds(PTAIL_BASE, PTAIL)])

    def _z(i, carry):
        zbuf_v[pl.ds(pl.multiple_of(i * 16, 16), 16)] = jnp.zeros((16,), _f32)
        return carry
    lax.fori_loop(0, ZCH // 16, _z, 0, unroll=8)
    pltpu.sync_copy(zbuf_v, tsh.at[pl.ds(zbase, ZCH)])
    pltpu.sync_copy(gix_hbm.at[pl.ds(ebase, EPT)], gix_v)
    pltpu.sync_copy(cix_hbm.at[pl.ds(ebase, EPT)], cix_v)
    plsc.subcore_barrier()
    pltpu.async_copy(psh.at[gix_v], vals_v, sem).wait()
    pltpu.sync_copy(vals_v, tsh.at[cix_v], add=True)
    plsc.subcore_barrier()
    pltpu.sync_copy(tsh.at[pl.ds(zbase, ZCH)], zbuf_v)
    pltpu.sync_copy(zbuf_v, tout_hbm.at[pl.ds(cid * TPAD + zbase, ZCH)])


_edge_pass = pl.kernel(
    _edge_body,
    out_type=jax.ShapeDtypeStruct((NC * TPAD,), _f32),
    mesh=plsc.VectorSubcoreMesh(core_axis_name="c", subcore_axis_name="s",
                                num_cores=NC, num_subcores=NS),
    scratch_types=[
        pltpu.VMEM((EPT,), jnp.int32),
        pltpu.VMEM((EPT,), jnp.int32),
        pltpu.VMEM((EPT,), _f32),
        pltpu.VMEM((ZCH,), _f32),
        pltpu.VMEM_SHARED((TPAD,), _f32),
        pltpu.VMEM_SHARED((TBINS,), _f32),
        pltpu.SemaphoreType.DMA,
    ],
)


def _stem_body(tab_hbm, idx_hbm, out_hbm, idx_v, vals_v, sem):
    cid = lax.axis_index("c")
    sid = lax.axis_index("s")
    wid = cid * NS + sid
    base = pl.multiple_of(wid * SPT, SPT)
    pltpu.sync_copy(idx_hbm.at[pl.ds(base, SPT)], idx_v)
    pltpu.async_copy(tab_hbm.at[idx_v], vals_v, sem).wait()
    pltpu.sync_copy(vals_v, out_hbm.at[pl.ds(base, SPT)])


_stem_gather = pl.kernel(
    _stem_body,
    out_type=jax.ShapeDtypeStruct((NSCAL,), _f32),
    mesh=plsc.VectorSubcoreMesh(core_axis_name="c", subcore_axis_name="s",
                                num_cores=NC, num_subcores=NS),
    scratch_types=[
        pltpu.VMEM((SPT,), jnp.int32),
        pltpu.VMEM((SPT,), _f32),
        pltpu.SemaphoreType.DMA,
    ],
)


def _init_body(xT_ref, batchT_ref, t0_ref, t1_ref, vecT_ref, bembT_blk_ref,
               w1aT_ref, w1bT_ref, b1_ref, w2T_ref, b2_ref, bemb_ref,
               h_ref, p_ref, invdeg_ref):
    oh_x = (xT_ref[...] == lax.broadcasted_iota(jnp.int32, (NBLK, NNODES), 0)
            ).astype(_f32)
    xw = _dot(w1aT_ref[...], bembT_blk_ref[...])
    oh_b = (batchT_ref[...] ==
            lax.broadcasted_iota(jnp.int32, (NGRAPHS, NNODES), 0)).astype(_f32)
    vb = _dot(w1bT_ref[...], vecT_ref[...])
    pre = _dot(xw, oh_x) + _dot(vb, oh_b) + b1_ref[...]
    h0 = _dot(w2T_ref[...], _lrelu(pre)) + b2_ref[...]
    h_ref[...] = h0
    p_ref[...] = _dot(bemb_ref[...], h0)
    deg = jnp.sum(t0_ref[...] + t1_ref[...], axis=0, keepdims=True)
    invdeg_ref[...] = 1.0 / jnp.maximum(deg, 1.0)


def _tc_init(xT, batchT, t0, t1, vecT, bembT_blk, w1aT, w1bT, b1, w2T, b2,
             bemb):
    return pl.pallas_call(
        _init_body,
        out_shape=(
            jax.ShapeDtypeStruct((NEMB, NNODES), _f32),
            jax.ShapeDtypeStruct((NBOND, NNODES), _f32),
            jax.ShapeDtypeStruct((1, NNODES), _f32),
        ),
    )(xT, batchT, t0, t1, vecT, bembT_blk, w1aT, w1bT, b1, w2T, b2, bemb)


def _step_body(t0_ref, t1_ref, invdeg_ref, h_ref, bembT_ref, rootT_ref,
               cb_ref, wihT_ref, bih_ref, whhT_ref, bhh_ref, bemb_ref,
               hout_ref, pout_ref):
    h = h_ref[...]
    t = t0_ref[...] + t1_ref[...]
    aggr = _dot(bembT_ref[...], t) * invdeg_ref[...]
    m = _lrelu(aggr + _dot(rootT_ref[...], h) + cb_ref[...])
    gi = _dot(wihT_ref[...], m) + bih_ref[...]
    gh = _dot(whhT_ref[...], h) + bhh_ref[...]
    r = jax.nn.sigmoid(gi[0:NEMB] + gh[0:NEMB])
    z = jax.nn.sigmoid(gi[NEMB:2 * NEMB] + gh[NEMB:2 * NEMB])
    n = jnp.tanh(gi[2 * NEMB:] + r * gh[2 * NEMB:])
    hn = (1.0 - z) * n + z * h
    hout_ref[...] = hn
    pout_ref[...] = _dot(bemb_ref[...], hn)


def _tc_step(t0, t1, invdeg, h, bembT, rootT, cb, wihT, bih, whhT, bhh, bemb):
    return pl.pallas_call(
        _step_body,
        out_shape=(
            jax.ShapeDtypeStruct((NEMB, NNODES), _f32),
            jax.ShapeDtypeStruct((NBOND, NNODES), _f32),
        ),
    )(t0, t1, invdeg, h, bembT, rootT, cb, wihT, bih, whhT, bhh, bemb)


def _head_body(rows_ref, stypesT_ref, batchT_ref, h_ref, stembT_ref,
               sw1aT_ref, sw1bT_ref, sb1_ref, sw2T_ref, sb2_ref, sw3_ref,
               sb3_ref, gw1_ref, gb1_ref, gw2_ref, gb2_ref,
               sp_ref, mp_ref):
    oh_st = (stypesT_ref[...] ==
             lax.broadcasted_iota(jnp.int32, (NST, 2048), 0)).astype(_f32)
    st = _dot(stembT_ref[...], oh_st)
    hs = _lrelu(_dot(sw1aT_ref[...], rows_ref[...])
                + _dot(sw1bT_ref[...], st) + sb1_ref[...])
    hs = _lrelu(_dot(sw2T_ref[...], hs) + sb2_ref[...])
    sp_ref[...] = lax.dot_general(
        hs, sw3_ref[...], (((0,), (0,)), ((), ())),
        preferred_element_type=_f32) + sb3_ref[...]
    oh_g = (batchT_ref[...] ==
            lax.broadcasted_iota(jnp.int32, (NGRAPHS, NNODES), 0)).astype(_f32)
    gsum = lax.dot_general(
        oh_g, h_ref[...], (((1,), (1,)), ((), ())),
        preferred_element_type=_f32)
    gcnt = jnp.sum(oh_g, axis=1, keepdims=True)
    gmean = gsum / jnp.maximum(gcnt, 1.0)
    gm = _lrelu(_dot(gmean, gw1_ref[...]) + gb1_ref[...])
    mp_ref[...] = _dot(gm, gw2_ref[...]) + gb2_ref[...]


def _tc_head(rows, stypesT, batchT, h, stembT, sw1aT, sw1bT, sb1, sw2T, sb2,
             sw3, sb3, gw1, gb1, gw2, gb2):
    return pl.pallas_call(
        _head_body,
        out_shape=(
            jax.ShapeDtypeStruct((2048, OUT_PER_STEM), _f32),
            jax.ShapeDtypeStruct((NGRAPHS, 1), _f32),
        ),
    )(rows, stypesT, batchT, h, stembT, sw1aT, sw1bT, sb1, sw2T, sb2,
      sw3, sb3, gw1, gb1, gw2, gb2)


def kernel(x, edge_index, edge_attr_idx, stemtypes, batch, stems_batch, stems,
           slices_x, vec_data, blockemb, stememb, bondemb, conv_root,
           conv_bias, b2e_w1, b2e_b1, b2e_w2, b2e_b2, gru_wih, gru_bih,
           gru_whh, gru_bhh, s_w1, s_b1, s_w2, s_b2, s_w3, s_b3, g_w1, g_b1,
           g_w2, g_b2):
    i32 = jnp.int32

    src = edge_index[0].astype(i32)
    dst = edge_index[1].astype(i32)
    i0 = edge_attr_idx[:, 0].astype(i32)
    i1 = edge_attr_idx[:, 1].astype(i32)
    gix = jnp.concatenate(
        [i0 * NNODES + src, jnp.zeros((EPAD - NEDGES,), i32)])
    cix = jnp.concatenate(
        [i1 * NNODES + dst, jnp.full((EPAD - NEDGES,), TBINS, i32)])
    ones_t = jnp.ones((TBINS,), _f32)

    sidx = (jnp.take(slices_x, stems_batch).astype(i32)
            + stems[:, 0].astype(i32))
    sidx_p = jnp.concatenate([sidx, jnp.zeros((2048 - NSTEMS,), i32)])
    sflat = (jnp.arange(NEMB, dtype=i32)[:, None] * NNODES
             + sidx_p[None, :]).reshape(NSCAL)
    stypesT = jnp.concatenate(
        [stemtypes.astype(i32), jnp.zeros((2048 - NSTEMS,), i32)]
    ).reshape(1, 2048)

    xT = x.astype(i32).reshape(1, NNODES)
    batchT = batch.astype(i32).reshape(1, NNODES)

    bembT = jnp.transpose(bondemb)
    w1aT = jnp.transpose(b2e_w1[:NEMB])
    w1bT = jnp.transpose(b2e_w1[NEMB:])
    vecT = jnp.transpose(vec_data)
    bembT_blk = jnp.transpose(blockemb)
    b1 = b2e_b1.reshape(NEMB, 1)
    w2T = jnp.transpose(b2e_w2)
    b2 = b2e_b2.reshape(NEMB, 1)
    rootT = jnp.transpose(conv_root)
    cb = conv_bias.reshape(NEMB, 1)
    wihT = jnp.transpose(gru_wih)
    whhT = jnp.transpose(gru_whh)
    bih = gru_bih.reshape(3 * NEMB, 1)
    bhh = gru_bhh.reshape(3 * NEMB, 1)
    sw1aT = jnp.transpose(s_w1[:NEMB])
    sw1bT = jnp.transpose(s_w1[NEMB:])
    sb1 = s_b1.reshape(NEMB, 1)
    sw2T = jnp.transpose(s_w2)
    sb2 = s_b2.reshape(NEMB, 1)
    sb3 = s_b3.reshape(1, OUT_PER_STEM)
    stembT = jnp.transpose(stememb)
    gb1 = g_b1.reshape(1, NEMB)
    gb2 = g_b2.reshape(1, 1)

    cpart = _edge_pass(ones_t, gix, cix)
    t0c = cpart[:TBINS].reshape(NBOND, NNODES)
    t1c = cpart[TPAD:TPAD + TBINS].reshape(NBOND, NNODES)

    h, p, invdeg = _tc_init(xT, batchT, t0c, t1c, vecT, bembT_blk,
                            w1aT, w1bT, b1, w2T, b2, bondemb)

    for _ in range(NSTEPS):
        tpart = _edge_pass(p.reshape(TBINS), gix, cix)
        t0 = tpart[:TBINS].reshape(NBOND, NNODES)
        t1 = tpart[TPAD:TPAD + TBINS].reshape(NBOND, NNODES)
        h, p = _tc_step(t0, t1, invdeg, h, bembT, rootT, cb,
                        wihT, bih, whhT, bhh, bondemb)

    rows = _stem_gather(h.reshape(NEMB * NNODES), sflat).reshape(NEMB, 2048)
    sp, mp = _tc_head(rows, stypesT, batchT, h, stembT, sw1aT, sw1bT, sb1,
                      sw2T, sb2, s_w3, sb3, g_w1, gb1, g_w2, gb2)
    return (sp[:NSTEMS], mp)
